# Initial kernel scaffold; baseline (speedup 1.0000x reference)
import jax
import jax.numpy as jnp
from jax.experimental import pallas as pl


def kernel(feat, edge_index, W1, b1, gamma1, beta1, a1, W2, b2, gamma2, beta2, a2):
    raise NotImplementedError("write your pallas kernel here")


# speedup vs baseline: 4.9877x; 4.9877x over previous
import functools

import jax
import jax.numpy as jnp
from jax import lax
from jax.experimental import pallas as pl
from jax.experimental.pallas import tpu as pltpu
from jax.experimental.pallas import tpu_sc as plsc

N_NODES = 10000
N_EDGES = 320000
IN_DIM = 128
HIDDEN = 256
OUT_DIM = 128
D = 128

NC, NS = 2, 16
NW = NC * NS
EPW = N_EDGES // NW
CHUNK = 80
NCHUNK = EPW // CHUNK
N_PAD = 10240
RPT = N_PAD // NS

_MESH = plsc.VectorSubcoreMesh(
    core_axis_name="c", subcore_axis_name="s", num_cores=NC, num_subcores=NS
)


@functools.partial(
    pl.kernel,
    out_type=jax.ShapeDtypeStruct((NC, 2, N_NODES), jnp.float32),
    mesh=_MESH,
    scratch_types=[
        pltpu.VMEM((CHUNK,), jnp.int32),
        pltpu.VMEM((CHUNK,), jnp.float32),
        pltpu.VMEM_SHARED((N_NODES,), jnp.float32),
        pltpu.VMEM_SHARED((N_NODES,), jnp.float32),
    ],
)
def _sc_degrees(src_hbm, dst_hbm, zeros_hbm, degs_hbm, idx_v, ones_v, sh_out, sh_in):
    c = lax.axis_index("c")
    s = lax.axis_index("s")
    for k in range(CHUNK // 16):
        ones_v[pl.ds(k * 16, 16)] = jnp.full((16,), 1.0, jnp.float32)

    @pl.when(s == 0)
    def _():
        pltpu.sync_copy(zeros_hbm, sh_out)

    @pl.when(s == 1)
    def _():
        pltpu.sync_copy(zeros_hbm, sh_in)

    plsc.subcore_barrier()
    base = (s * NC + c) * EPW

    def step(i, carry):
        off = base + i * CHUNK
        pltpu.sync_copy(src_hbm.at[pl.ds(off, CHUNK)], idx_v)
        pltpu.sync_copy(ones_v, sh_out.at[idx_v], add=True)
        pltpu.sync_copy(dst_hbm.at[pl.ds(off, CHUNK)], idx_v)
        pltpu.sync_copy(ones_v, sh_in.at[idx_v], add=True)
        return carry

    lax.fori_loop(0, NCHUNK, step, 0)
    plsc.subcore_barrier()

    @pl.when(s == 0)
    def _():
        pltpu.sync_copy(sh_out, degs_hbm.at[c, 0])

    @pl.when(s == 1)
    def _():
        pltpu.sync_copy(sh_in, degs_hbm.at[c, 1])


@functools.partial(
    pl.kernel,
    out_type=jax.ShapeDtypeStruct((NC, N_PAD, D), jnp.float32),
    mesh=_MESH,
    scratch_types=[
        pltpu.VMEM((CHUNK,), jnp.int32),
        pltpu.VMEM((CHUNK,), jnp.int32),
        pltpu.VMEM((CHUNK, D), jnp.float32),
        pltpu.VMEM_SHARED((N_PAD, D), jnp.float32),
        pltpu.SemaphoreType.DMA,
    ],
)
def _sc_edge_pass(table_hbm, src_hbm, dst_hbm, zrows_hbm, parts_hbm,
                  isrc_v, idst_v, rows_v, accum_sh, sem):
    c = lax.axis_index("c")
    s = lax.axis_index("s")
    r0 = s * RPT
    pltpu.sync_copy(zrows_hbm.at[pl.ds(r0, RPT)], accum_sh.at[pl.ds(r0, RPT)])
    plsc.subcore_barrier()
    base = (s * NC + c) * EPW

    def step(i, carry):
        off = base + i * CHUNK
        pltpu.sync_copy(src_hbm.at[pl.ds(off, CHUNK)], isrc_v)
        pltpu.sync_copy(dst_hbm.at[pl.ds(off, CHUNK)], idst_v)
        pltpu.async_copy(table_hbm.at[isrc_v], rows_v, sem).wait()
        pltpu.sync_copy(rows_v, accum_sh.at[idst_v], add=True)
        return carry

    lax.fori_loop(0, NCHUNK, step, 0)
    plsc.subcore_barrier()
    pltpu.sync_copy(accum_sh.at[pl.ds(r0, RPT)], parts_hbm.at[c, pl.ds(r0, RPT)])


def _tc_prep_body(degs_ref, feat_ref, dout_ref, din_ref, hs_ref):
    deg_o = degs_ref[0, 0] + degs_ref[1, 0]
    deg_i = degs_ref[0, 1] + degs_ref[1, 1]
    dout = lax.rsqrt(jnp.maximum(deg_o, 1.0))
    din = lax.rsqrt(jnp.maximum(deg_i, 1.0))
    dout_ref[...] = dout
    din_ref[...] = din
    hs_ref[...] = feat_ref[...] * dout[:, None]


_tc_prep = pl.pallas_call(
    _tc_prep_body,
    out_shape=(
        jax.ShapeDtypeStruct((N_NODES,), jnp.float32),
        jax.ShapeDtypeStruct((N_NODES,), jnp.float32),
        jax.ShapeDtypeStruct((N_NODES, D), jnp.float32),
    ),
)


def _tc_mid_body(p_ref, din_ref, dout_ref, W1_ref, b1_ref, g1_ref, be1_ref,
                 a1_ref, W2_ref, t2_ref):
    agg = (p_ref[0, :N_NODES] + p_ref[1, :N_NODES]) * din_ref[...][:, None]
    z = jnp.dot(agg, W1_ref[...], preferred_element_type=jnp.float32) + b1_ref[...]
    mean = jnp.mean(z, axis=0)
    zc = z - mean
    var = jnp.mean(zc * zc, axis=0)
    zn = zc * lax.rsqrt(var + 1e-5) * g1_ref[...] + be1_ref[...]
    h1 = jnp.where(zn > 0, zn, zn * a1_ref[...])
    hs2 = h1 * dout_ref[...][:, None]
    t2_ref[...] = jnp.dot(hs2, W2_ref[...], preferred_element_type=jnp.float32)


_tc_mid = pl.pallas_call(
    _tc_mid_body,
    out_shape=jax.ShapeDtypeStruct((N_NODES, D), jnp.float32),
)


def _tc_out_body(p_ref, din_ref, b2_ref, g2_ref, be2_ref, a2_ref, out_ref):
    z = (p_ref[0, :N_NODES] + p_ref[1, :N_NODES]) * din_ref[...][:, None] + b2_ref[...]
    mean = jnp.mean(z, axis=0)
    zc = z - mean
    var = jnp.mean(zc * zc, axis=0)
    zn = zc * lax.rsqrt(var + 1e-5) * g2_ref[...] + be2_ref[...]
    out_ref[...] = jnp.where(zn > 0, zn, zn * a2_ref[...])


_tc_out = pl.pallas_call(
    _tc_out_body,
    out_shape=jax.ShapeDtypeStruct((N_NODES, OUT_DIM), jnp.float32),
)


def kernel(feat, edge_index, W1, b1, gamma1, beta1, a1, W2, b2, gamma2, beta2, a2):
    src = edge_index[0].astype(jnp.int32)
    dst = edge_index[1].astype(jnp.int32)
    zeros1 = jnp.zeros((N_NODES,), jnp.float32)
    zrows = jnp.zeros((N_PAD, D), jnp.float32)

    degs = _sc_degrees(src, dst, zeros1)
    dout_inv, din_inv, hscaled = _tc_prep(degs, feat)
    p1 = _sc_edge_pass(hscaled, src, dst, zrows)
    t2 = _tc_mid(p1, din_inv, dout_inv, W1, b1, gamma1, beta1,
                 a1.reshape(1, 1), W2)
    p2 = _sc_edge_pass(t2, src, dst, zrows)
    return _tc_out(p2, din_inv, b2, gamma2, beta2, a2.reshape(1, 1))
